# baseline (speedup 1.0000x reference)
import jax
import jax.numpy as jnp
from jax import lax
from jax.experimental import pallas as pl
from jax.experimental.pallas import tpu as pltpu
from jax.experimental.pallas import tpu_sc as plsc

B = 16384
F = 26
FP = 32
NW = 32
RPW = B // NW
CH = 16
GPC = CH * FP // 128
NCH = RPW // CH
IRW = RPW * FP // 128


def _fm_body(sidx_hbm, idx_hbm, emb_hbm, lin_hbm, bias_hbm, out_hbm,
             sidx_v, idx_v, emb_v, lin_v, out_v, bias_v, sem):
    w = lax.axis_index("s") * 2 + lax.axis_index("c")
    lanes = lax.iota(jnp.int32, 16)
    lane0 = lanes == 0
    tail_mask = lanes < (F - 16)

    pltpu.sync_copy(sidx_hbm.at[pl.ds(w * IRW, IRW)], sidx_v)
    pltpu.sync_copy(idx_hbm.at[pl.ds(w * IRW, IRW)], idx_v)
    pltpu.sync_copy(bias_hbm, bias_v.at[pl.ds(0, 1)])
    bias_lane0 = jnp.where(lane0, bias_v[...], 0.0)

    def chunk_body(c, carry):
        copies = []
        for j in range(GPC):
            g = c * GPC + j
            copies.append(pltpu.make_async_copy(
                emb_hbm.at[sidx_v.at[g]], emb_v.at[pl.ds(j * 128, 128)], sem))
            copies.append(pltpu.make_async_copy(
                lin_hbm.at[idx_v.at[g]], lin_v.at[pl.ds(j * 128, 128)], sem))
        for cp in copies:
            cp.start()
        for cp in copies:
            cp.wait()

        def row_body(b, carry2):
            iv0 = idx_v[c * GPC + (b >> 2), pl.ds((b & 3) * FP, 16)]
            iv1 = idx_v[c * GPC + (b >> 2), pl.ds((b & 3) * FP + 16, 16)]
            ov0 = (iv0 & 3) << 5
            ov1 = (iv1 & 3) << 5
            s0 = jnp.zeros((16,), jnp.float32)
            s1 = jnp.zeros((16,), jnp.float32)
            q0 = jnp.zeros((16,), jnp.float32)
            q1 = jnp.zeros((16,), jnp.float32)
            for f in range(F):
                o = ov0[f] if f < 16 else ov1[f - 16]
                r = b * FP + f
                v0 = emb_v[r, pl.ds(o, 16)]
                v1 = emb_v[r, pl.ds(o + 16, 16)]
                s0 = s0 + v0
                s1 = s1 + v1
                q0 = q0 + v0 * v0
                q1 = q1 + v1 * v1
            lv0 = lin_v[pl.ds(b * FP, 16)]
            lv1 = jnp.where(tail_mask, lin_v[pl.ds(b * FP + 16, 16)], 0.0)
            t = (s0 * s0 - q0 + s1 * s1 - q1) * 0.5
            total = jnp.sum(t + lv0 + lv1 + bias_lane0)
            plsc.store_scatter(out_v, [jnp.broadcast_to(c * CH + b, (16,))],
                               jnp.broadcast_to(total, (16,)), mask=lane0)
            return carry2

        lax.fori_loop(0, CH, row_body, 0)
        return carry

    lax.fori_loop(0, NCH, chunk_body, 0)
    pltpu.sync_copy(out_v, out_hbm.at[pl.ds(w * RPW, RPW)])


VOC = 1000000
LBLK = 2000
NLBLK = VOC // LBLK


def kernel(interaction_pairs, embedding_weight, linear_weight, bias):
    idxp = jnp.concatenate(
        [interaction_pairs, interaction_pairs[:, :FP - F]], axis=1)
    idx32 = idxp.reshape(-1, 128)
    sidx = (idxp >> 2).reshape(-1, 128)
    emb4 = embedding_weight.reshape(-1, 128)
    lin1 = linear_weight.T.reshape(-1)
    run = pl.kernel(
        _fm_body,
        out_type=jax.ShapeDtypeStruct((B,), jnp.float32),
        mesh=plsc.VectorSubcoreMesh(core_axis_name="c", subcore_axis_name="s"),
        compiler_params=pltpu.CompilerParams(needs_layout_passes=False),
        scratch_types=[
            pltpu.VMEM((IRW, 128), jnp.int32),
            pltpu.VMEM((IRW, 128), jnp.int32),
            pltpu.VMEM((CH * FP, 128), jnp.float32),
            pltpu.VMEM((CH * FP,), jnp.float32),
            pltpu.VMEM((RPW,), jnp.float32),
            pltpu.VMEM((16,), jnp.float32),
            pltpu.SemaphoreType.DMA,
        ],
    )
    return run(sidx, idx32, emb4, lin1, bias)


# speedup vs baseline: 1.0009x; 1.0009x over previous
import jax
import jax.numpy as jnp
from jax import lax
from jax.experimental import pallas as pl
from jax.experimental.pallas import tpu as pltpu
from jax.experimental.pallas import tpu_sc as plsc

B = 16384
F = 26
FP = 32
NW = 32
RPW = B // NW
CH = 16
GPC = CH * FP // 128
NCH = RPW // CH
IRW = RPW * FP // 128


def _fm_body(sidx_hbm, idx_hbm, emb_hbm, lin_hbm, bias_hbm, out_hbm,
             sidx_v, idx_v, emb_v, lin_v, out_v, bias_v, sem):
    w = lax.axis_index("s") * 2 + lax.axis_index("c")
    lanes = lax.iota(jnp.int32, 16)
    lane0 = lanes == 0
    tail_mask = lanes < (F - 16)

    pltpu.sync_copy(sidx_hbm.at[pl.ds(w * IRW, IRW)], sidx_v)
    pltpu.sync_copy(idx_hbm.at[pl.ds(w * IRW, IRW)], idx_v)
    pltpu.sync_copy(bias_hbm, bias_v.at[pl.ds(0, 1)])
    bias_lane0 = jnp.where(lane0, bias_v[...], 0.0)

    def chunk_body(c, carry):
        copies = []
        for j in range(GPC):
            g = c * GPC + j
            copies.append(pltpu.make_async_copy(
                emb_hbm.at[sidx_v.at[g]], emb_v.at[pl.ds(j * 128, 128)], sem))
            copies.append(pltpu.make_async_copy(
                lin_hbm.at[idx_v.at[g]], lin_v.at[pl.ds(j * 128, 128)], sem))
        for cp in copies:
            cp.start()
        for cp in copies:
            cp.wait()

        def row_body(b, carry2):
            iv0 = idx_v[c * GPC + (b >> 2), pl.ds((b & 3) * FP, 16)]
            iv1 = idx_v[c * GPC + (b >> 2), pl.ds((b & 3) * FP + 16, 16)]
            ov0 = (iv0 & 3) << 5
            ov1 = (iv1 & 3) << 5
            s0 = jnp.zeros((16,), jnp.float32)
            s1 = jnp.zeros((16,), jnp.float32)
            q0 = jnp.zeros((16,), jnp.float32)
            q1 = jnp.zeros((16,), jnp.float32)
            for f in range(F):
                o = ov0[f] if f < 16 else ov1[f - 16]
                r = b * FP + f
                v0 = emb_v[r, pl.ds(o, 16)]
                v1 = emb_v[r, pl.ds(o + 16, 16)]
                s0 = s0 + v0
                s1 = s1 + v1
                q0 = q0 + v0 * v0
                q1 = q1 + v1 * v1
            lv0 = lin_v[pl.ds(b * FP, 16)]
            lv1 = jnp.where(tail_mask, lin_v[pl.ds(b * FP + 16, 16)], 0.0)
            t = (s0 * s0 - q0 + s1 * s1 - q1) * 0.5
            total = jnp.sum(t + lv0 + lv1 + bias_lane0)
            plsc.store_scatter(out_v, [jnp.broadcast_to(c * CH + b, (16,))],
                               jnp.broadcast_to(total, (16,)), mask=lane0)
            return carry2

        lax.fori_loop(0, CH, row_body, 0)
        return carry

    lax.fori_loop(0, NCH, chunk_body, 0)
    pltpu.sync_copy(out_v, out_hbm.at[pl.ds(w * RPW, RPW)])


VOC = 1000000
LBLK = 2000
NLBLK = VOC // LBLK


def kernel(interaction_pairs, embedding_weight, linear_weight, bias):
    idxp = jnp.concatenate(
        [interaction_pairs, interaction_pairs[:, :FP - F]], axis=1)
    idx32 = idxp.reshape(-1, 128)
    sidx = (idxp >> 2).reshape(-1, 128)
    emb4 = embedding_weight.reshape(-1, 128)
    lin1 = jnp.sum(linear_weight, axis=1)
    run = pl.kernel(
        _fm_body,
        out_type=jax.ShapeDtypeStruct((B,), jnp.float32),
        mesh=plsc.VectorSubcoreMesh(core_axis_name="c", subcore_axis_name="s"),
        compiler_params=pltpu.CompilerParams(needs_layout_passes=False),
        scratch_types=[
            pltpu.VMEM((IRW, 128), jnp.int32),
            pltpu.VMEM((IRW, 128), jnp.int32),
            pltpu.VMEM((CH * FP, 128), jnp.float32),
            pltpu.VMEM((CH * FP,), jnp.float32),
            pltpu.VMEM((RPW,), jnp.float32),
            pltpu.VMEM((16,), jnp.float32),
            pltpu.SemaphoreType.DMA,
        ],
    )
    return run(sidx, idx32, emb4, lin1, bias)
